# baseline (speedup 1.0000x reference)
import functools

import jax
import jax.numpy as jnp
from jax import lax
from jax.experimental import pallas as pl
from jax.experimental.pallas import tpu as pltpu
from jax.experimental.pallas import tpu_sc as plsc

N = 10000
E = 320000
D = 128
NC = 2
NS = 16
NW = NC * NS
EPW = E // NW
K = 48
NB = 4
NCHF = EPW // K
TAIL = EPW - NCHF * K
NACC = 10240
RPT = NACC // NS
L = 16


def _prep_body(x_ref, a_ref, y_ref, wn_ref):
    x = x_ref[...]
    a = a_ref[:D]
    u = jnp.dot(x, a, preferred_element_type=jnp.float32)
    wn = jnp.exp(u - jnp.max(u))
    y_ref[...] = x * wn
    wn_ref[...] = wn


def _sc_body(ei_hbm, y_hbm, wn_hbm, outg_hbm, outs_hbm,
             sidx_v, didx_v, dscat_v, sidxt_v, didxt_v, rows_v, wn_v, s_v,
             g_sh, gsem0, gsem1, gsem2, gsem3, ssem0, ssem1,
             isem0, isem1, isem2, isem3):
    cid = lax.axis_index("c")
    sid = lax.axis_index("s")
    wid = sid * NC + cid
    tile = sid
    gsem = (gsem0, gsem1, gsem2, gsem3)
    ssem = (ssem0, ssem1)
    isem = (isem0, isem1, isem2, isem3)

    pltpu.sync_copy(wn_hbm, wn_v)
    zeros = jnp.zeros((L,), jnp.float32)

    def zrow(r):
        for j in range(D // L):
            rows_v[0, r, pl.ds(j * L, L)] = zeros
    pl.loop(0, K)(zrow)

    def zs(i):
        s_v[pl.ds(i * L, L)] = zeros
    pl.loop(0, N // L)(zs)

    for b in range(RPT // K):
        pltpu.sync_copy(rows_v.at[0],
                        g_sh.at[pl.ds(tile * RPT + b * K, K)])
    pltpu.sync_copy(rows_v.at[0, pl.ds(0, RPT - (RPT // K) * K)],
                    g_sh.at[pl.ds(tile * RPT + (RPT // K) * K,
                                  RPT - (RPT // K) * K)])
    plsc.subcore_barrier()

    base = wid * EPW

    def ig(i, p4):
        e0 = base + i * K
        pltpu.async_copy(ei_hbm.at[pl.ds(e0, K)], sidx_v.at[p4], isem[p4])
        pltpu.async_copy(ei_hbm.at[pl.ds(E + e0, K)], didx_v.at[p4],
                         isem[p4])

    def iw(p4):
        pltpu.make_async_copy(ei_hbm.at[pl.ds(base, K)], sidx_v.at[p4],
                              isem[p4]).wait()
        pltpu.make_async_copy(ei_hbm.at[pl.ds(base, K)], didx_v.at[p4],
                              isem[p4]).wait()

    def g(p4):
        pltpu.async_copy(y_hbm.at[sidx_v.at[p4]], rows_v.at[p4], gsem[p4])

    def gw(p4):
        pltpu.make_async_copy(y_hbm.at[sidx_v.at[p4]], rows_v.at[p4],
                              gsem[p4]).wait()

    def cidx(p4, p2):
        for t in range(K // L):
            dscat_v[p2, pl.ds(t * L, L)] = didx_v[p4, pl.ds(t * L, L)]

    def s(p4, p2):
        pltpu.async_copy(rows_v.at[p4], g_sh.at[dscat_v.at[p2]], ssem[p2],
                         add=True)

    def sw(p4, p2):
        pltpu.make_async_copy(rows_v.at[p4], g_sh.at[dscat_v.at[p2]],
                              ssem[p2]).wait()

    def scal(p4):
        for t in range(K // L):
            si = sidx_v[p4, pl.ds(t * L, L)]
            di = didx_v[p4, pl.ds(t * L, L)]
            w16 = plsc.load_gather(wn_v, [si])
            plsc.addupdate_scatter(s_v, [di], w16)

    for b in range(NB):
        ig(b, b)
    for b in range(NB - 1):
        iw(b)
        g(b)
    gw(0)
    iw(3)
    g(3)
    cidx(0, 0)
    s(0, 0)
    scal(0)
    ig(4, 0)

    def steady(j):
        for b in range(4):
            i = 1 + 4 * j + b
            p4 = (1 + b) % 4
            p2 = (1 + b) % 2
            gw(p4)
            sw((p4 + 3) % 4, (p2 + 1) % 2)
            iw((p4 + 3) % 4)
            g((p4 + 3) % 4)
            cidx(p4, p2)
            s(p4, p2)
            scal(p4)
            ig(i + 4, p4)
    pl.loop(0, 50)(steady)

    for i in range(201, NCHF):
        p4 = i % 4
        p2 = i % 2
        gw(p4)
        sw((p4 + 3) % 4, (p2 + 1) % 2)
        if i + 3 < NCHF:
            iw((p4 + 3) % 4)
            g((p4 + 3) % 4)
        cidx(p4, p2)
        s(p4, p2)
        scal(p4)
        if i + 4 < NCHF:
            ig(i + 4, p4)
    sw((NCHF - 1) % 4, (NCHF - 1) % 2)

    t0 = base + NCHF * K
    pltpu.sync_copy(ei_hbm.at[pl.ds(t0, TAIL)], sidxt_v)
    pltpu.sync_copy(ei_hbm.at[pl.ds(E + t0, TAIL)], didxt_v)
    pltpu.async_copy(y_hbm.at[sidxt_v], rows_v.at[0, pl.ds(0, TAIL)],
                     gsem[0]).wait()
    pltpu.sync_copy(rows_v.at[0, pl.ds(0, TAIL)], g_sh.at[didxt_v], add=True)
    wt = plsc.load_gather(wn_v, [sidxt_v[...]])
    plsc.addupdate_scatter(s_v, [didxt_v[...]], wt)
    plsc.subcore_barrier()

    pltpu.sync_copy(g_sh.at[pl.ds(tile * RPT, RPT)],
                    outg_hbm.at[cid, pl.ds(tile * RPT, RPT)])
    pltpu.sync_copy(s_v, outs_hbm.at[wid])


_sc_aggregate = functools.partial(
    pl.kernel,
    out_type=(
        jax.ShapeDtypeStruct((NC, NACC, D), jnp.float32),
        jax.ShapeDtypeStruct((NW, N), jnp.float32),
    ),
    mesh=plsc.VectorSubcoreMesh(core_axis_name="c", subcore_axis_name="s"),
    compiler_params=pltpu.CompilerParams(needs_layout_passes=False),
    scratch_types=[
        pltpu.VMEM((NB, K), jnp.int32),
        pltpu.VMEM((NB, K), jnp.int32),
        pltpu.VMEM((2, K), jnp.int32),
        pltpu.VMEM((TAIL,), jnp.int32),
        pltpu.VMEM((TAIL,), jnp.int32),
        pltpu.VMEM((NB, K, D), jnp.float32),
        pltpu.VMEM((N,), jnp.float32),
        pltpu.VMEM((N,), jnp.float32),
        pltpu.VMEM_SHARED((NACC, D), jnp.float32),
        pltpu.SemaphoreType.DMA,
        pltpu.SemaphoreType.DMA,
        pltpu.SemaphoreType.DMA,
        pltpu.SemaphoreType.DMA,
        pltpu.SemaphoreType.DMA,
        pltpu.SemaphoreType.DMA,
        pltpu.SemaphoreType.DMA,
        pltpu.SemaphoreType.DMA,
        pltpu.SemaphoreType.DMA,
        pltpu.SemaphoreType.DMA,
    ],
)(_sc_body)


_RB = 1024


def _combine_body(gp_ref, sp_ref, x_ref, wf_ref, bf_ref, o_ref):
    g = gp_ref[0] + gp_ref[1]
    s = jnp.sum(sp_ref[...], axis=1, keepdims=True)
    h = jnp.where(s > 0.0,
                  g / jnp.maximum(s, 1e-16),
                  x_ref[...])
    dn = (((1,), (1,)), ((), ()))
    o_ref[:, :D] = lax.dot_general(
        h, wf_ref[0], dn, preferred_element_type=jnp.float32) + bf_ref[0]
    o_ref[:, D:] = lax.dot_general(
        h, wf_ref[1], dn, preferred_element_type=jnp.float32) + bf_ref[1]


def kernel(x, edge_index, Wf, bf, Wa, ba):
    a_sum = (Wa[0] + Wa[1]).reshape(2 * D, 1)

    y, wn = pl.pallas_call(
        _prep_body,
        out_shape=(
            jax.ShapeDtypeStruct((N, D), jnp.float32),
            jax.ShapeDtypeStruct((N, 1), jnp.float32),
        ),
    )(x, a_sum)

    gp, sp = _sc_aggregate(edge_index.reshape(2 * E), y, wn.reshape(N))

    out = pl.pallas_call(
        _combine_body,
        grid=(NACC // _RB,),
        in_specs=[
            pl.BlockSpec((NC, _RB, D), lambda i: (0, i, 0)),
            pl.BlockSpec((_RB, NW), lambda i: (i, 0)),
            pl.BlockSpec((_RB, D), lambda i: (i, 0)),
            pl.BlockSpec((2, D, D), lambda i: (0, 0, 0)),
            pl.BlockSpec((2, D), lambda i: (0, 0)),
        ],
        out_specs=pl.BlockSpec((_RB, 2 * D), lambda i: (i, 0)),
        out_shape=jax.ShapeDtypeStruct((N, 2 * D), jnp.float32),
    )(gp, sp.T, x, Wf, bf)
    return out


# speedup vs baseline: 1.1149x; 1.1149x over previous
import functools

import jax
import jax.numpy as jnp
from jax import lax
from jax.experimental import pallas as pl
from jax.experimental.pallas import tpu as pltpu
from jax.experimental.pallas import tpu_sc as plsc

N = 10000
E = 320000
D = 128
NC = 2
NS = 16
NW = NC * NS
EPW = E // NW
K = 64
NCHF = EPW // K
TAIL = EPW - NCHF * K
NACC = 10240
RPT = NACC // NS
L = 16


def _prep_body(x_ref, a_ref, y_ref, wn_ref):
    x = x_ref[...]
    a = a_ref[:D]
    u = jnp.dot(x, a, preferred_element_type=jnp.float32)
    wn = jnp.exp(u - jnp.max(u))
    y_ref[...] = x * wn
    wn_ref[...] = wn


def _sc_body(ei_hbm, y_hbm, wn_hbm, outg_hbm, outs_hbm,
             sidx_v, didx_v, dscat_v, sidxt_v, didxt_v, rows_v, wn_v, s_v,
             g_sh, gsem0, gsem1, gsem2, ssem0, ssem1, isem0, isem1, isem2):
    cid = lax.axis_index("c")
    sid = lax.axis_index("s")
    wid = sid * NC + cid
    tile = sid
    gsem = (gsem0, gsem1, gsem2)
    ssem = (ssem0, ssem1)
    isem = (isem0, isem1, isem2)

    pltpu.sync_copy(wn_hbm, wn_v)
    zeros = jnp.zeros((L,), jnp.float32)

    def zrow(r):
        for j in range(D // L):
            rows_v[0, r, pl.ds(j * L, L)] = zeros
    pl.loop(0, K)(zrow)

    def zs(i):
        s_v[pl.ds(i * L, L)] = zeros
    pl.loop(0, N // L)(zs)

    for b in range(RPT // K):
        pltpu.sync_copy(rows_v.at[0],
                        g_sh.at[pl.ds(tile * RPT + b * K, K)])
    plsc.subcore_barrier()

    base = wid * EPW

    def ig(i, p3):
        e0 = base + i * K
        pltpu.async_copy(ei_hbm.at[pl.ds(e0, K)], sidx_v.at[p3], isem[p3])
        pltpu.async_copy(ei_hbm.at[pl.ds(E + e0, K)], didx_v.at[p3],
                         isem[p3])

    def iw(p3):
        pltpu.make_async_copy(ei_hbm.at[pl.ds(base, K)], sidx_v.at[p3],
                              isem[p3]).wait()
        pltpu.make_async_copy(ei_hbm.at[pl.ds(base, K)], didx_v.at[p3],
                              isem[p3]).wait()

    def g(p3):
        pltpu.async_copy(y_hbm.at[sidx_v.at[p3]], rows_v.at[p3], gsem[p3])

    def gw(p3):
        pltpu.make_async_copy(y_hbm.at[sidx_v.at[p3]], rows_v.at[p3],
                              gsem[p3]).wait()

    def cidx(p3, p2):
        for t in range(K // L):
            dscat_v[p2, pl.ds(t * L, L)] = didx_v[p3, pl.ds(t * L, L)]

    def s(p3, p2):
        pltpu.async_copy(rows_v.at[p3], g_sh.at[dscat_v.at[p2]], ssem[p2],
                         add=True)

    def sw(p3, p2):
        pltpu.make_async_copy(rows_v.at[p3], g_sh.at[dscat_v.at[p2]],
                              ssem[p2]).wait()

    def scal(p3):
        for t in range(K // L):
            si = sidx_v[p3, pl.ds(t * L, L)]
            di = didx_v[p3, pl.ds(t * L, L)]
            w16 = plsc.load_gather(wn_v, [si])
            plsc.addupdate_scatter(s_v, [di], w16)

    ig(0, 0)
    ig(1, 1)
    ig(2, 2)
    iw(0)
    g(0)
    iw(1)
    g(1)
    gw(0)
    iw(2)
    g(2)
    cidx(0, 0)
    s(0, 0)
    scal(0)
    ig(3, 0)

    def steady(j):
        for b in range(6):
            i = 1 + 6 * j + b
            p3 = (1 + b) % 3
            p2 = (1 + b) % 2
            gw(p3)
            sw((p3 + 2) % 3, (p2 + 1) % 2)
            iw((p3 + 2) % 3)
            g((p3 + 2) % 3)
            cidx(p3, p2)
            s(p3, p2)
            scal(p3)
            ig(i + 3, p3)
    pl.loop(0, 25)(steady)

    for i in range(151, NCHF):
        p3 = i % 3
        p2 = i % 2
        gw(p3)
        sw((p3 + 2) % 3, (p2 + 1) % 2)
        if i + 2 < NCHF:
            iw((p3 + 2) % 3)
            g((p3 + 2) % 3)
        cidx(p3, p2)
        s(p3, p2)
        scal(p3)
        if i + 3 < NCHF:
            ig(i + 3, p3)
    sw((NCHF - 1) % 3, (NCHF - 1) % 2)

    t0 = base + NCHF * K
    pltpu.sync_copy(ei_hbm.at[pl.ds(t0, TAIL)], sidxt_v)
    pltpu.sync_copy(ei_hbm.at[pl.ds(E + t0, TAIL)], didxt_v)
    pltpu.async_copy(y_hbm.at[sidxt_v], rows_v.at[0, pl.ds(0, TAIL)],
                     gsem[0]).wait()
    pltpu.sync_copy(rows_v.at[0, pl.ds(0, TAIL)], g_sh.at[didxt_v], add=True)
    wt = plsc.load_gather(wn_v, [sidxt_v[...]])
    plsc.addupdate_scatter(s_v, [didxt_v[...]], wt)
    plsc.subcore_barrier()

    pltpu.sync_copy(g_sh.at[pl.ds(tile * RPT, RPT)],
                    outg_hbm.at[cid, pl.ds(tile * RPT, RPT)])
    pltpu.sync_copy(s_v, outs_hbm.at[wid])


_sc_aggregate = functools.partial(
    pl.kernel,
    out_type=(
        jax.ShapeDtypeStruct((NC, NACC, D), jnp.float32),
        jax.ShapeDtypeStruct((NW, N), jnp.float32),
    ),
    mesh=plsc.VectorSubcoreMesh(core_axis_name="c", subcore_axis_name="s"),
    compiler_params=pltpu.CompilerParams(needs_layout_passes=False),
    scratch_types=[
        pltpu.VMEM((3, K), jnp.int32),
        pltpu.VMEM((3, K), jnp.int32),
        pltpu.VMEM((2, K), jnp.int32),
        pltpu.VMEM((TAIL,), jnp.int32),
        pltpu.VMEM((TAIL,), jnp.int32),
        pltpu.VMEM((3, K, D), jnp.float32),
        pltpu.VMEM((N,), jnp.float32),
        pltpu.VMEM((N,), jnp.float32),
        pltpu.VMEM_SHARED((NACC, D), jnp.float32),
        pltpu.SemaphoreType.DMA,
        pltpu.SemaphoreType.DMA,
        pltpu.SemaphoreType.DMA,
        pltpu.SemaphoreType.DMA,
        pltpu.SemaphoreType.DMA,
        pltpu.SemaphoreType.DMA,
        pltpu.SemaphoreType.DMA,
        pltpu.SemaphoreType.DMA,
    ],
)(_sc_body)


_RB = 1024


def _combine_body(gp_ref, sp_ref, x_ref, wf_ref, bf_ref, o_ref):
    g = gp_ref[0] + gp_ref[1]
    ones = jnp.ones((NW, 1), jnp.float32)
    s = lax.dot_general(sp_ref[...], ones, (((0,), (0,)), ((), ())),
                        preferred_element_type=jnp.float32)
    h = jnp.where(s > 0.0,
                  g / jnp.maximum(s, 1e-16),
                  x_ref[...])
    dn = (((1,), (1,)), ((), ()))
    o_ref[:, :D] = lax.dot_general(
        h, wf_ref[0], dn, preferred_element_type=jnp.float32) + bf_ref[0]
    o_ref[:, D:] = lax.dot_general(
        h, wf_ref[1], dn, preferred_element_type=jnp.float32) + bf_ref[1]


def kernel(x, edge_index, Wf, bf, Wa, ba):
    a_sum = (Wa[0] + Wa[1]).reshape(2 * D, 1)

    y, wn = pl.pallas_call(
        _prep_body,
        out_shape=(
            jax.ShapeDtypeStruct((N, D), jnp.float32),
            jax.ShapeDtypeStruct((N, 1), jnp.float32),
        ),
    )(x, a_sum)

    gp, sp = _sc_aggregate(edge_index.reshape(2 * E), y, wn.reshape(N))

    out = pl.pallas_call(
        _combine_body,
        grid=(NACC // _RB,),
        in_specs=[
            pl.BlockSpec((NC, _RB, D), lambda i: (0, i, 0)),
            pl.BlockSpec((NW, _RB), lambda i: (0, i)),
            pl.BlockSpec((_RB, D), lambda i: (i, 0)),
            pl.BlockSpec((2, D, D), lambda i: (0, 0, 0)),
            pl.BlockSpec((2, D), lambda i: (0, 0)),
        ],
        out_specs=pl.BlockSpec((_RB, 2 * D), lambda i: (i, 0)),
        out_shape=jax.ShapeDtypeStruct((N, 2 * D), jnp.float32),
    )(gp, sp, x, Wf, bf)
    return out
